# baseline (speedup 1.0000x reference)
import functools

import jax
import jax.numpy as jnp
from jax import lax
from jax.experimental import pallas as pl
from jax.experimental.pallas import tpu as pltpu
from jax.experimental.pallas import tpu_sc as plsc

D = 64
HIST = 50
BATCH = 16384
VOCAB = 1000000
NC = 2
NS = 16
NW = NC * NS
IPW = BATCH // NW
CH = 8
L = CH * HIST
NCHUNK = IPW // CH
NBUF = 2
NGRP = NCHUNK // NBUF

RB = 512
HALF = RB // 2
PBLK = -(-VOCAB // RB)
PROWS = PBLK * HALF


def _relayout_body(wt_ref, out_ref):
    x = wt_ref[...]
    ident = jax.lax.broadcasted_iota(jnp.int32, (D, D), 0)
    ident = (ident == jax.lax.broadcasted_iota(jnp.int32, (D, D), 1)).astype(jnp.float32)
    y = jax.lax.dot_general(x, ident, (((0,), (0,)), ((), ())),
                            preferred_element_type=jnp.float32)
    out_ref[:, :D] = y[:HALF]
    out_ref[:, D:] = y[HALF:]


_relayout = pl.pallas_call(
    _relayout_body,
    grid=(PBLK,),
    in_specs=[pl.BlockSpec((D, RB), lambda j: (0, j))],
    out_specs=pl.BlockSpec((HALF, 2 * D), lambda j: (j, 0)),
    out_shape=jax.ShapeDtypeStruct((PROWS, 2 * D), jnp.float32),
)


@functools.partial(
    pl.kernel,
    out_type=jax.ShapeDtypeStruct((BATCH, HIST, D), jnp.float32),
    mesh=plsc.VectorSubcoreMesh(core_axis_name="c", subcore_axis_name="s"),
    compiler_params=pltpu.CompilerParams(use_tc_tiling_on_sc=False),
    scratch_types=(
        [pltpu.VMEM((NCHUNK, L), jnp.int32)]
        + [pltpu.VMEM((L, D), jnp.float32) for _ in range(NBUF)]
        + [pltpu.SemaphoreType.DMA for _ in range(2 * NBUF)]
    ),
)
def _emb_lookup(idx_hbm, table_hbm, out_hbm, idx_v, *bufs):
    rows = bufs[:NBUF]
    gsem = bufs[NBUF:2 * NBUF]
    wsem = bufs[2 * NBUF:]
    wid = lax.axis_index("s") * NC + lax.axis_index("c")
    ibase = wid * IPW
    pltpu.sync_copy(idx_hbm.at[wid], idx_v)

    def gdesc(c, b):
        return table_hbm.at[idx_v.at[c]], rows[b]

    def wdescs(c, b):
        return [(rows[b].at[pl.ds(m * HIST, HIST)],
                 out_hbm.at[ibase + c * CH + m]) for m in range(CH)]

    for b in range(NBUF):
        src, dst = gdesc(b, b)
        pltpu.async_copy(src, dst, gsem[b])

    def grp_body(grp, carry):
        for b in range(NBUF):
            c = grp * NBUF + b
            gs, gd = gdesc(c, b)
            pltpu.make_async_copy(gs, gd, gsem[b]).wait()
            wpairs = wdescs(c, b)
            for ws, wd in wpairs:
                pltpu.async_copy(ws, wd, wsem[b])

            @pl.when(grp < NGRP - 1)
            def _():
                for ws, wd in wpairs:
                    pltpu.make_async_copy(ws, wd, wsem[b]).wait()
                ns, nd = gdesc(c + NBUF, b)
                pltpu.async_copy(ns, nd, gsem[b])

        return carry

    lax.fori_loop(0, NGRP, grp_body, 0)

    for b in range(NBUF):
        for ws, wd in wdescs(NCHUNK - NBUF + b, b):
            pltpu.make_async_copy(ws, wd, wsem[b]).wait()


def kernel(input_ids, weight):
    table = _relayout(weight.T).reshape(2 * PROWS, D)
    idx = input_ids.astype(jnp.int32)
    o = idx % RB
    idx = idx - o + 2 * (o % HALF) + o // HALF
    idx = idx.reshape(NW, NCHUNK, L)
    return _emb_lookup(idx, table)


# speedup vs baseline: 1.9383x; 1.9383x over previous
import functools

import jax
import jax.numpy as jnp
from jax import lax
from jax.experimental import pallas as pl
from jax.experimental.pallas import tpu as pltpu
from jax.experimental.pallas import tpu_sc as plsc

D = 64
HIST = 50
BATCH = 16384
VOCAB = 1000000
NC = 2
NS = 16
NW = NC * NS
IPW = BATCH // NW
CH = 8
L = CH * HIST
NCHUNK = IPW // CH
NBUF = 2
NGRP = NCHUNK // NBUF

RB = 4096
HALF = RB // 2
PBLK = -(-VOCAB // RB)
PROWS = PBLK * HALF


def _relayout_body(wt_ref, out_ref):
    y = wt_ref[...].T
    out_ref[:, :D] = y[:HALF]
    out_ref[:, D:] = y[HALF:]


_relayout = pl.pallas_call(
    _relayout_body,
    grid=(PBLK,),
    in_specs=[pl.BlockSpec((D, RB), lambda j: (0, j))],
    out_specs=pl.BlockSpec((HALF, 2 * D), lambda j: (j, 0)),
    out_shape=jax.ShapeDtypeStruct((PROWS, 2 * D), jnp.float32),
)


@functools.partial(
    pl.kernel,
    out_type=jax.ShapeDtypeStruct((BATCH, HIST, D), jnp.float32),
    mesh=plsc.VectorSubcoreMesh(core_axis_name="c", subcore_axis_name="s"),
    compiler_params=pltpu.CompilerParams(use_tc_tiling_on_sc=False),
    scratch_types=(
        [pltpu.VMEM((NCHUNK, L), jnp.int32)]
        + [pltpu.VMEM((L, D), jnp.float32) for _ in range(NBUF)]
        + [pltpu.SemaphoreType.DMA for _ in range(2 * NBUF)]
    ),
)
def _emb_lookup(idx_hbm, table_hbm, out_hbm, idx_v, *bufs):
    rows = bufs[:NBUF]
    gsem = bufs[NBUF:2 * NBUF]
    wsem = bufs[2 * NBUF:]
    wid = lax.axis_index("s") * NC + lax.axis_index("c")
    ibase = wid * IPW
    pltpu.sync_copy(idx_hbm.at[wid], idx_v)

    def gdesc(c, b):
        return table_hbm.at[idx_v.at[c]], rows[b]

    def wdescs(c, b):
        return [(rows[b].at[pl.ds(m * HIST, HIST)],
                 out_hbm.at[ibase + c * CH + m]) for m in range(CH)]

    for b in range(NBUF):
        src, dst = gdesc(b, b)
        pltpu.async_copy(src, dst, gsem[b])

    def grp_body(grp, carry):
        for b in range(NBUF):
            c = grp * NBUF + b
            gs, gd = gdesc(c, b)
            pltpu.make_async_copy(gs, gd, gsem[b]).wait()
            wpairs = wdescs(c, b)
            for ws, wd in wpairs:
                pltpu.async_copy(ws, wd, wsem[b])

            @pl.when(grp < NGRP - 1)
            def _():
                for ws, wd in wpairs:
                    pltpu.make_async_copy(ws, wd, wsem[b]).wait()
                ns, nd = gdesc(c + NBUF, b)
                pltpu.async_copy(ns, nd, gsem[b])

        return carry

    lax.fori_loop(0, NGRP, grp_body, 0)

    for b in range(NBUF):
        for ws, wd in wdescs(NCHUNK - NBUF + b, b):
            pltpu.make_async_copy(ws, wd, wsem[b]).wait()


def kernel(input_ids, weight):
    table = _relayout(weight.T).reshape(2 * PROWS, D)
    idx = input_ids.astype(jnp.int32)
    o = idx % RB
    idx = idx - o + 2 * (o % HALF) + o // HALF
    idx = idx.reshape(NW, NCHUNK, L)
    return _emb_lookup(idx, table)


# speedup vs baseline: 2.0820x; 1.0741x over previous
import functools

import jax
import jax.numpy as jnp
from jax import lax
from jax.experimental import pallas as pl
from jax.experimental.pallas import tpu as pltpu
from jax.experimental.pallas import tpu_sc as plsc

D = 64
HIST = 50
BATCH = 16384
VOCAB = 1000000
NC = 2
NS = 16
NW = NC * NS
IPW = BATCH // NW
CH = 8
L = CH * HIST
NCHUNK = IPW // CH
NBUF = 2
NGRP = NCHUNK // NBUF

RB = 8192
HALF = RB // 2
PBLK = -(-VOCAB // RB)
PROWS = PBLK * HALF


def _relayout_body(wt_ref, out_ref):
    y = wt_ref[...].T
    out_ref[:, :D] = y[:HALF]
    out_ref[:, D:] = y[HALF:]


_relayout = pl.pallas_call(
    _relayout_body,
    grid=(PBLK,),
    in_specs=[pl.BlockSpec((D, RB), lambda j: (0, j))],
    out_specs=pl.BlockSpec((HALF, 2 * D), lambda j: (j, 0)),
    out_shape=jax.ShapeDtypeStruct((PROWS, 2 * D), jnp.float32),
)


@functools.partial(
    pl.kernel,
    out_type=jax.ShapeDtypeStruct((BATCH, HIST, D), jnp.float32),
    mesh=plsc.VectorSubcoreMesh(core_axis_name="c", subcore_axis_name="s"),
    compiler_params=pltpu.CompilerParams(use_tc_tiling_on_sc=False),
    scratch_types=(
        [pltpu.VMEM((NCHUNK, L), jnp.int32)]
        + [pltpu.VMEM((L, D), jnp.float32) for _ in range(NBUF)]
        + [pltpu.SemaphoreType.DMA for _ in range(2 * NBUF)]
    ),
)
def _emb_lookup(idx_hbm, table_hbm, out_hbm, idx_v, *bufs):
    rows = bufs[:NBUF]
    gsem = bufs[NBUF:2 * NBUF]
    wsem = bufs[2 * NBUF:]
    wid = lax.axis_index("s") * NC + lax.axis_index("c")
    ibase = wid * IPW
    pltpu.sync_copy(idx_hbm.at[wid], idx_v)

    def gdesc(c, b):
        return table_hbm.at[idx_v.at[c]], rows[b]

    def wdescs(c, b):
        return [(rows[b].at[pl.ds(m * HIST, HIST)],
                 out_hbm.at[ibase + c * CH + m]) for m in range(CH)]

    for b in range(NBUF):
        src, dst = gdesc(b, b)
        pltpu.async_copy(src, dst, gsem[b])

    def grp_body(grp, carry):
        for b in range(NBUF):
            c = grp * NBUF + b
            gs, gd = gdesc(c, b)
            pltpu.make_async_copy(gs, gd, gsem[b]).wait()
            wpairs = wdescs(c, b)
            for ws, wd in wpairs:
                pltpu.async_copy(ws, wd, wsem[b])

            @pl.when(grp < NGRP - 1)
            def _():
                for ws, wd in wpairs:
                    pltpu.make_async_copy(ws, wd, wsem[b]).wait()
                ns, nd = gdesc(c + NBUF, b)
                pltpu.async_copy(ns, nd, gsem[b])

        return carry

    lax.fori_loop(0, NGRP, grp_body, 0)

    for b in range(NBUF):
        for ws, wd in wdescs(NCHUNK - NBUF + b, b):
            pltpu.make_async_copy(ws, wd, wsem[b]).wait()


def kernel(input_ids, weight):
    table = _relayout(weight.T).reshape(2 * PROWS, D)
    idx = input_ids.astype(jnp.int32)
    o = idx % RB
    idx = idx - o + 2 * (o % HALF) + o // HALF
    idx = idx.reshape(NW, NCHUNK, L)
    return _emb_lookup(idx, table)


# speedup vs baseline: 2.1477x; 1.0316x over previous
import functools

import jax
import jax.numpy as jnp
from jax import lax
from jax.experimental import pallas as pl
from jax.experimental.pallas import tpu as pltpu
from jax.experimental.pallas import tpu_sc as plsc

D = 64
HIST = 50
BATCH = 16384
VOCAB = 1000000
NC = 2
NS = 16
NW = NC * NS
IPW = BATCH // NW
CH = 8
L = CH * HIST
NCHUNK = IPW // CH
NBUF = 2
NGRP = NCHUNK // NBUF

RB = 16384
HALF = RB // 2
PBLK = -(-VOCAB // RB)
PROWS = PBLK * HALF


def _relayout_body(wt_ref, out_ref):
    y = wt_ref[...].T
    out_ref[:, :D] = y[:HALF]
    out_ref[:, D:] = y[HALF:]


_relayout = pl.pallas_call(
    _relayout_body,
    grid=(PBLK,),
    in_specs=[pl.BlockSpec((D, RB), lambda j: (0, j))],
    out_specs=pl.BlockSpec((HALF, 2 * D), lambda j: (j, 0)),
    out_shape=jax.ShapeDtypeStruct((PROWS, 2 * D), jnp.float32),
)


@functools.partial(
    pl.kernel,
    out_type=jax.ShapeDtypeStruct((BATCH, HIST, D), jnp.float32),
    mesh=plsc.VectorSubcoreMesh(core_axis_name="c", subcore_axis_name="s"),
    compiler_params=pltpu.CompilerParams(use_tc_tiling_on_sc=False),
    scratch_types=(
        [pltpu.VMEM((NCHUNK, L), jnp.int32)]
        + [pltpu.VMEM((L, D), jnp.float32) for _ in range(NBUF)]
        + [pltpu.SemaphoreType.DMA for _ in range(2 * NBUF)]
    ),
)
def _emb_lookup(idx_hbm, table_hbm, out_hbm, idx_v, *bufs):
    rows = bufs[:NBUF]
    gsem = bufs[NBUF:2 * NBUF]
    wsem = bufs[2 * NBUF:]
    wid = lax.axis_index("s") * NC + lax.axis_index("c")
    ibase = wid * IPW
    pltpu.sync_copy(idx_hbm.at[wid], idx_v)

    def gdesc(c, b):
        return table_hbm.at[idx_v.at[c]], rows[b]

    def wdescs(c, b):
        return [(rows[b].at[pl.ds(m * HIST, HIST)],
                 out_hbm.at[ibase + c * CH + m]) for m in range(CH)]

    for b in range(NBUF):
        src, dst = gdesc(b, b)
        pltpu.async_copy(src, dst, gsem[b])

    def grp_body(grp, carry):
        for b in range(NBUF):
            c = grp * NBUF + b
            gs, gd = gdesc(c, b)
            pltpu.make_async_copy(gs, gd, gsem[b]).wait()
            wpairs = wdescs(c, b)
            for ws, wd in wpairs:
                pltpu.async_copy(ws, wd, wsem[b])

            @pl.when(grp < NGRP - 1)
            def _():
                for ws, wd in wpairs:
                    pltpu.make_async_copy(ws, wd, wsem[b]).wait()
                ns, nd = gdesc(c + NBUF, b)
                pltpu.async_copy(ns, nd, gsem[b])

        return carry

    lax.fori_loop(0, NGRP, grp_body, 0)

    for b in range(NBUF):
        for ws, wd in wdescs(NCHUNK - NBUF + b, b):
            pltpu.make_async_copy(ws, wd, wsem[b]).wait()


def kernel(input_ids, weight):
    table = _relayout(weight.T).reshape(2 * PROWS, D)
    idx = input_ids.astype(jnp.int32)
    o = idx % RB
    idx = idx - o + 2 * (o % HALF) + o // HALF
    idx = idx.reshape(NW, NCHUNK, L)
    return _emb_lookup(idx, table)
